# baseline (speedup 1.0000x reference)
import functools

import jax
import jax.numpy as jnp
from jax import lax
from jax.experimental import pallas as pl
from jax.experimental.pallas import tpu as pltpu
from jax.experimental.pallas import tpu_sc as plsc

NC = 2
NS = 16
NW = NC * NS
ROWS_PER_BUF = 2
H_SPLITS = (104, 96)


@functools.lru_cache(maxsize=None)
def _make_gather(V, D, Bt, H):
    assert Bt % (NW * ROWS_PER_BUF) == 0 and H <= 128 and H % 8 == 0
    r_per_w = Bt // NW
    mesh = plsc.VectorSubcoreMesh(core_axis_name="c", subcore_axis_name="s")

    @functools.partial(
        pl.kernel,
        out_type=jax.ShapeDtypeStruct((Bt, H, D), jnp.float32),
        mesh=mesh,
        scratch_types=[
            pltpu.VMEM((r_per_w, H), jnp.int32),
            pltpu.VMEM((ROWS_PER_BUF, H, D), jnp.float32),
            pltpu.VMEM((ROWS_PER_BUF, H, D), jnp.float32),
            pltpu.SemaphoreType.DMA,
            pltpu.SemaphoreType.DMA,
            pltpu.SemaphoreType.DMA,
            pltpu.SemaphoreType.DMA,
        ],
        compiler_params=pltpu.CompilerParams(use_tc_tiling_on_sc=False),
    )
    def gather_kernel(table_hbm, x_hbm, out_hbm, idx_v, rows0, rows1,
                      gsem0, gsem1, ssem0, ssem1):
        wid = lax.axis_index("s") * NC + lax.axis_index("c")
        base = wid * r_per_w
        pltpu.sync_copy(x_hbm.at[pl.ds(base, r_per_w)], idx_v)

        def fire_gathers(i, rows, gsem):
            return [pltpu.async_copy(table_hbm.at[idx_v.at[i + r]],
                                     rows.at[r], gsem)
                    for r in range(ROWS_PER_BUF)]

        @pl.loop(0, r_per_w, step=2 * ROWS_PER_BUF)
        def _(i):
            g0 = fire_gathers(i, rows0, gsem0)
            g1 = fire_gathers(i + ROWS_PER_BUF, rows1, gsem1)
            for cp in g0:
                cp.wait()
            s0 = pltpu.async_copy(
                rows0, out_hbm.at[pl.ds(base + i, ROWS_PER_BUF)], ssem0)
            for cp in g1:
                cp.wait()
            s1 = pltpu.async_copy(
                rows1,
                out_hbm.at[pl.ds(base + i + ROWS_PER_BUF, ROWS_PER_BUF)],
                ssem1)
            s0.wait()
            s1.wait()

    return gather_kernel


def kernel(x, pe_weight):
    Bt, H = x.shape
    V, D = pe_weight.shape
    xi = x.astype(jnp.int32)
    parts = []
    off = 0
    for h in H_SPLITS:
        parts.append(_make_gather(V, D, Bt, h)(pe_weight,
                                               xi[:, off:off + h]))
        off += h
    return jnp.concatenate(parts, axis=1)


# speedup vs baseline: 1.1974x; 1.1974x over previous
import functools

import jax
import jax.numpy as jnp
from jax import lax
from jax.experimental import pallas as pl
from jax.experimental.pallas import tpu as pltpu
from jax.experimental.pallas import tpu_sc as plsc

NC = 2
NS = 16
NW = NC * NS
ROWS_PER_BUF = 4
SPLIT = (104, 96)


@functools.lru_cache(maxsize=None)
def _make_gather(V, D, Bt, H):
    assert Bt % (NW * ROWS_PER_BUF) == 0 and sum(SPLIT) == H
    r_per_w = Bt // NW
    mesh = plsc.VectorSubcoreMesh(core_axis_name="c", subcore_axis_name="s")

    @functools.partial(
        pl.kernel,
        out_type=jax.ShapeDtypeStruct((Bt, H, D), jnp.float32),
        mesh=mesh,
        scratch_types=[
            pltpu.VMEM((r_per_w, H), jnp.int32),
            pltpu.VMEM((ROWS_PER_BUF, H, D), jnp.float32),
            pltpu.VMEM((ROWS_PER_BUF, H, D), jnp.float32),
            pltpu.SemaphoreType.DMA,
            pltpu.SemaphoreType.DMA,
            pltpu.SemaphoreType.DMA,
            pltpu.SemaphoreType.DMA,
        ],
        compiler_params=pltpu.CompilerParams(use_tc_tiling_on_sc=False),
    )
    def gather_kernel(table_hbm, x_hbm, out_hbm, idx_v, rows0, rows1,
                      gsem0, gsem1, ssem0, ssem1):
        wid = lax.axis_index("s") * NC + lax.axis_index("c")
        base = wid * r_per_w
        pltpu.sync_copy(x_hbm.at[pl.ds(base, r_per_w)], idx_v)

        def fire_gathers(i, rows, gsem):
            cps = []
            for r in range(ROWS_PER_BUF):
                off = 0
                for w in SPLIT:
                    cps.append(pltpu.async_copy(
                        table_hbm.at[idx_v.at[i + r, pl.ds(off, w)]],
                        rows.at[r, pl.ds(off, w)], gsem))
                    off += w
            return cps

        @pl.loop(0, r_per_w, step=2 * ROWS_PER_BUF)
        def _(i):
            g0 = fire_gathers(i, rows0, gsem0)
            g1 = fire_gathers(i + ROWS_PER_BUF, rows1, gsem1)
            for cp in g0:
                cp.wait()
            s0 = pltpu.async_copy(
                rows0, out_hbm.at[pl.ds(base + i, ROWS_PER_BUF)], ssem0)
            for cp in g1:
                cp.wait()
            s1 = pltpu.async_copy(
                rows1,
                out_hbm.at[pl.ds(base + i + ROWS_PER_BUF, ROWS_PER_BUF)],
                ssem1)
            s0.wait()
            s1.wait()

    return gather_kernel


def kernel(x, pe_weight):
    Bt, H = x.shape
    V, D = pe_weight.shape
    return _make_gather(V, D, Bt, H)(pe_weight, x.astype(jnp.int32))


# speedup vs baseline: 1.2006x; 1.0027x over previous
import functools

import jax
import jax.numpy as jnp
from jax import lax
from jax.experimental import pallas as pl
from jax.experimental.pallas import tpu as pltpu
from jax.experimental.pallas import tpu_sc as plsc

NC = 2
NS = 16
NW = NC * NS
ROWS_PER_BUF = 4
SPLIT = (200,)


@functools.lru_cache(maxsize=None)
def _make_gather(V, D, Bt, H):
    assert Bt % (NW * ROWS_PER_BUF) == 0 and sum(SPLIT) == H
    r_per_w = Bt // NW
    mesh = plsc.VectorSubcoreMesh(core_axis_name="c", subcore_axis_name="s")

    @functools.partial(
        pl.kernel,
        out_type=jax.ShapeDtypeStruct((Bt, H, D), jnp.float32),
        mesh=mesh,
        scratch_types=[
            pltpu.VMEM((r_per_w, H), jnp.int32),
            pltpu.VMEM((ROWS_PER_BUF, H, D), jnp.float32),
            pltpu.VMEM((ROWS_PER_BUF, H, D), jnp.float32),
            pltpu.SemaphoreType.DMA,
            pltpu.SemaphoreType.DMA,
            pltpu.SemaphoreType.DMA,
            pltpu.SemaphoreType.DMA,
        ],
        compiler_params=pltpu.CompilerParams(use_tc_tiling_on_sc=False),
    )
    def gather_kernel(table_hbm, x_hbm, out_hbm, idx_v, rows0, rows1,
                      gsem0, gsem1, ssem0, ssem1):
        wid = lax.axis_index("s") * NC + lax.axis_index("c")
        base = wid * r_per_w
        pltpu.sync_copy(x_hbm.at[pl.ds(base, r_per_w)], idx_v)

        def fire_gathers(i, rows, gsem):
            cps = []
            for r in range(ROWS_PER_BUF):
                off = 0
                for w in SPLIT:
                    cps.append(pltpu.async_copy(
                        table_hbm.at[idx_v.at[i + r, pl.ds(off, w)]],
                        rows.at[r, pl.ds(off, w)], gsem))
                    off += w
            return cps

        @pl.loop(0, r_per_w, step=2 * ROWS_PER_BUF)
        def _(i):
            g0 = fire_gathers(i, rows0, gsem0)
            g1 = fire_gathers(i + ROWS_PER_BUF, rows1, gsem1)
            for cp in g0:
                cp.wait()
            s0 = pltpu.async_copy(
                rows0, out_hbm.at[pl.ds(base + i, ROWS_PER_BUF)], ssem0)
            for cp in g1:
                cp.wait()
            s1 = pltpu.async_copy(
                rows1,
                out_hbm.at[pl.ds(base + i + ROWS_PER_BUF, ROWS_PER_BUF)],
                ssem1)
            s0.wait()
            s1.wait()

    return gather_kernel


def kernel(x, pe_weight):
    Bt, H = x.shape
    V, D = pe_weight.shape
    return _make_gather(V, D, Bt, H)(pe_weight, x.astype(jnp.int32))


# speedup vs baseline: 1.2029x; 1.0019x over previous
import functools

import jax
import jax.numpy as jnp
from jax import lax
from jax.experimental import pallas as pl
from jax.experimental.pallas import tpu as pltpu
from jax.experimental.pallas import tpu_sc as plsc

NC = 2
NS = 16
NW = NC * NS
ROWS_PER_BUF = 4
SPLIT = (104, 96)


@functools.lru_cache(maxsize=None)
def _make_gather(V, D, Bt, H):
    assert Bt % (NW * ROWS_PER_BUF) == 0 and sum(SPLIT) == H
    r_per_w = Bt // NW
    mesh = plsc.VectorSubcoreMesh(core_axis_name="c", subcore_axis_name="s")

    @functools.partial(
        pl.kernel,
        out_type=jax.ShapeDtypeStruct((Bt, H, D), jnp.float32),
        mesh=mesh,
        scratch_types=[
            pltpu.VMEM((r_per_w, H), jnp.int32),
            pltpu.VMEM((ROWS_PER_BUF, H, D), jnp.float32),
            pltpu.VMEM((ROWS_PER_BUF, H, D), jnp.float32),
            pltpu.SemaphoreType.DMA,
            pltpu.SemaphoreType.DMA,
            pltpu.SemaphoreType.DMA,
            pltpu.SemaphoreType.DMA,
        ],
        compiler_params=pltpu.CompilerParams(use_tc_tiling_on_sc=False),
    )
    def gather_kernel(table_hbm, x_hbm, out_hbm, idx_v, rows0, rows1,
                      gsem0, gsem1, ssem0, ssem1):
        wid = lax.axis_index("s") * NC + lax.axis_index("c")
        base = wid * r_per_w
        pltpu.sync_copy(x_hbm.at[pl.ds(base, r_per_w)], idx_v)

        def fire_gathers(i, rows, gsem):
            cps = []
            for r in range(ROWS_PER_BUF):
                off = 0
                for w in SPLIT:
                    cps.append(pltpu.async_copy(
                        table_hbm.at[idx_v.at[i + r, pl.ds(off, w)]],
                        rows.at[r, pl.ds(off, w)], gsem))
                    off += w
            return cps

        @pl.loop(0, r_per_w, step=2 * ROWS_PER_BUF)
        def _(i):
            g0 = fire_gathers(i, rows0, gsem0)
            g1 = fire_gathers(i + ROWS_PER_BUF, rows1, gsem1)
            for cp in g0:
                cp.wait()
            s0 = pltpu.async_copy(
                rows0, out_hbm.at[pl.ds(base + i, ROWS_PER_BUF)], ssem0)
            for cp in g1:
                cp.wait()
            s1 = pltpu.async_copy(
                rows1,
                out_hbm.at[pl.ds(base + i + ROWS_PER_BUF, ROWS_PER_BUF)],
                ssem1)
            s0.wait()
            s1.wait()

    return gather_kernel


def kernel(x, pe_weight):
    Bt, H = x.shape
    V, D = pe_weight.shape
    return _make_gather(V, D, Bt, H)(pe_weight, x.astype(jnp.int32))
